# Initial kernel scaffold; baseline (speedup 1.0000x reference)
import jax
import jax.numpy as jnp
from jax.experimental import pallas as pl


def kernel(node_states, edge_index, rel_idx, rel_emb, msg_W1, msg_b1, msg_W2, msg_b2, gate_W1, gate_b1, gate_W2, gate_b2, ln_gamma, ln_beta):
    raise NotImplementedError("write your pallas kernel here")


# speedup vs baseline: 2.3675x; 2.3675x over previous
import functools

import jax
import jax.numpy as jnp
from jax import lax
from jax.experimental import pallas as pl
from jax.experimental.pallas import tpu as pltpu
from jax.experimental.pallas import tpu_sc as plsc

N = 10000
D = 128
E = 320000
NREL = 6

NC = 2
NS = 16
NW = NC * NS
IDXB = 128
ROWS_PER_W = 80
EDGES_PER_W = ROWS_PER_W * IDXB
E_PAD = NW * EDGES_PER_W
GROUP_ROWS = 4
GROUP_E = GROUP_ROWS * IDXB
N_GROUPS = ROWS_PER_W // GROUP_ROWS
SGROUP_ROWS = 2
SGROUP_E = SGROUP_ROWS * IDXB
SN_GROUPS = ROWS_PER_W // SGROUP_ROWS
ZROWS = 64

TE = 2048
TN = 2000
N_PAD = 10240
NPW = N_PAD // NS


def _node_transform_body(h_ref, wa_ref, ga_ref, s_ref, zd_ref):
    x = h_ref[...]
    s_ref[...] = jnp.dot(x, wa_ref[...], preferred_element_type=jnp.float32)
    zd_ref[...] = jnp.dot(x, ga_ref[...], preferred_element_type=jnp.float32)


def _node_transform(h, wa, ga):
    return pl.pallas_call(
        _node_transform_body,
        grid=(N // TN,),
        in_specs=[
            pl.BlockSpec((TN, D), lambda i: (i, 0)),
            pl.BlockSpec((D, D), lambda i: (0, 0)),
            pl.BlockSpec((D, D), lambda i: (0, 0)),
        ],
        out_specs=[
            pl.BlockSpec((TN, D), lambda i: (i, 0)),
            pl.BlockSpec((TN, D), lambda i: (i, 0)),
        ],
        out_shape=[
            jax.ShapeDtypeStruct((N, D), jnp.float32),
            jax.ShapeDtypeStruct((N, D), jnp.float32),
        ],
    )(h, wa, ga)


def _gelu(x):
    return 0.5 * x * (1.0 + lax.erf(x * 0.7071067811865476))


def _edge_mlp_body(src_ref, dst_ref, rel_ref, reltab_ref, w2_ref, b2_ref,
                   g1b_ref, g2_ref, gb2_ref, out_ref):
    pid = pl.program_id(0)
    rel = rel_ref[0, 0, :]
    onehot = jnp.where(
        lax.broadcasted_iota(jnp.int32, (TE, 8), 1) == rel[:, None],
        1.0, 0.0).astype(jnp.float32)
    relb = jnp.dot(onehot, reltab_ref[...],
                   preferred_element_type=jnp.float32)
    hid = _gelu(src_ref[...] + relb[:, :D])
    msg = jnp.dot(hid, w2_ref[...],
                  preferred_element_type=jnp.float32) + b2_ref[...]
    g1 = dst_ref[...] + jnp.dot(msg, g1b_ref[...],
                                preferred_element_type=jnp.float32) \
        + relb[:, D:]
    ghid = _gelu(g1)
    glog = jnp.sum(ghid * g2_ref[...], axis=1, keepdims=True) + gb2_ref[0, 0]
    out = jax.nn.sigmoid(glog) * msg
    eid = pid * TE + lax.broadcasted_iota(jnp.int32, (TE, 1), 0)
    out_ref[...] = jnp.where(eid < E, out, 0.0)


def _edge_mlp(src_g, dst_g, rel3d, reltab, w2, b2, g1b, g2row, gb2):
    return pl.pallas_call(
        _edge_mlp_body,
        grid=(E_PAD // TE,),
        in_specs=[
            pl.BlockSpec((TE, D), lambda i: (i, 0)),
            pl.BlockSpec((TE, D), lambda i: (i, 0)),
            pl.BlockSpec((1, 1, TE), lambda i: (i, 0, 0)),
            pl.BlockSpec((8, 2 * D), lambda i: (0, 0)),
            pl.BlockSpec((D, D), lambda i: (0, 0)),
            pl.BlockSpec((1, D), lambda i: (0, 0)),
            pl.BlockSpec((D, D), lambda i: (0, 0)),
            pl.BlockSpec((1, D), lambda i: (0, 0)),
            pl.BlockSpec((1, 1), lambda i: (0, 0), memory_space=pltpu.SMEM),
        ],
        out_specs=pl.BlockSpec((TE, D), lambda i: (i, 0)),
        out_shape=jax.ShapeDtypeStruct((E_PAD, D), jnp.float32),
    )(src_g, dst_g, rel3d, reltab, w2, b2, g1b, g2row, gb2)


def _ln_body(h_ref, p0_ref, p1_ref, g_ref, b_ref, out_ref):
    x = h_ref[...] + p0_ref[...] + p1_ref[...]
    m = jnp.mean(x, axis=-1, keepdims=True)
    xc = x - m
    v = jnp.mean(xc * xc, axis=-1, keepdims=True)
    out_ref[...] = xc * lax.rsqrt(v + 1e-5) * g_ref[...] + b_ref[...]


def _residual_ln(h, p0, p1, gamma, beta):
    return pl.pallas_call(
        _ln_body,
        grid=(N // TN,),
        in_specs=[
            pl.BlockSpec((TN, D), lambda i: (i, 0)),
            pl.BlockSpec((TN, D), lambda i: (i, 0)),
            pl.BlockSpec((TN, D), lambda i: (i, 0)),
            pl.BlockSpec((1, D), lambda i: (0, 0)),
            pl.BlockSpec((1, D), lambda i: (0, 0)),
        ],
        out_specs=pl.BlockSpec((TN, D), lambda i: (i, 0)),
        out_shape=jax.ShapeDtypeStruct((N, D), jnp.float32),
    )(h, p0, p1, gamma, beta)


@functools.cache
def _sc_mesh():
    return plsc.VectorSubcoreMesh(core_axis_name="c", subcore_axis_name="s",
                                  num_cores=NC, num_subcores=NS)


def _gather_kernel_body(s_hbm, zd_hbm, sidx_hbm, didx_hbm, srcg_hbm,
                        dstg_hbm, idx_v, rows_v, sem):
    cid = lax.axis_index("c")
    sid = lax.axis_index("s")
    wid = sid * NC + cid
    row0 = wid * ROWS_PER_W
    e0 = wid * EDGES_PER_W

    def run(table_hbm, idx2d_hbm, out_hbm):
        def group(g, carry):
            r = row0 + g * GROUP_ROWS
            pltpu.sync_copy(idx2d_hbm.at[pl.ds(r, GROUP_ROWS)], idx_v)
            descs = []
            for j in range(GROUP_ROWS):
                descs.append(pltpu.async_copy(
                    table_hbm.at[idx_v.at[j]],
                    rows_v.at[pl.ds(j * IDXB, IDXB)], sem))
            for d in descs:
                d.wait()
            pltpu.sync_copy(
                rows_v, out_hbm.at[pl.ds(e0 + g * GROUP_E, GROUP_E)])
            return carry
        lax.fori_loop(0, N_GROUPS, group, 0)

    run(s_hbm, sidx_hbm, srcg_hbm)
    run(zd_hbm, didx_hbm, dstg_hbm)


@functools.cache
def _sc_gather_kernel():
    return pl.kernel(
        _gather_kernel_body,
        out_type=[
            jax.ShapeDtypeStruct((E_PAD, D), jnp.float32),
            jax.ShapeDtypeStruct((E_PAD, D), jnp.float32),
        ],
        mesh=_sc_mesh(),
        scratch_types=[
            pltpu.VMEM((GROUP_ROWS, IDXB), jnp.int32),
            pltpu.VMEM((GROUP_E, D), jnp.float32),
            pltpu.SemaphoreType.DMA,
        ],
        compiler_params=pltpu.CompilerParams(use_tc_tiling_on_sc=False),
    )


def _sc_gather(s, zd, sidx, didx):
    return _sc_gather_kernel()(s, zd, sidx, didx)


def _scatter_kernel_body(eout_hbm, didx_hbm, out0_hbm, out1_hbm, idx_v,
                         vals_v, zbuf_v, acc_shared):
    cid = lax.axis_index("c")
    sid = lax.axis_index("s")
    wid = sid * NC + cid
    row0 = wid * ROWS_PER_W
    e0 = wid * EDGES_PER_W

    def zrow(i, carry):
        for j in range(D // 16):
            zbuf_v[i, pl.ds(j * 16, 16)] = jnp.zeros((16,), jnp.float32)
        return carry
    lax.fori_loop(0, ZROWS, zrow, 0)
    for t in range(NPW // ZROWS):
        pltpu.sync_copy(zbuf_v,
                        acc_shared.at[pl.ds(sid * NPW + t * ZROWS, ZROWS)])
    plsc.subcore_barrier()

    def group(g, carry):
        r = row0 + g * SGROUP_ROWS
        pltpu.sync_copy(didx_hbm.at[pl.ds(r, SGROUP_ROWS)], idx_v)
        pltpu.sync_copy(eout_hbm.at[pl.ds(e0 + g * SGROUP_E, SGROUP_E)],
                        vals_v)
        for j in range(SGROUP_ROWS):
            pltpu.sync_copy(vals_v.at[pl.ds(j * IDXB, IDXB)],
                            acc_shared.at[idx_v.at[j]], add=True)
        return carry
    lax.fori_loop(0, SN_GROUPS, group, 0)
    plsc.subcore_barrier()

    @pl.when(cid == 0)
    def _():
        pltpu.sync_copy(acc_shared.at[pl.ds(sid * NPW, NPW)],
                        out0_hbm.at[pl.ds(sid * NPW, NPW)])

    @pl.when(cid == 1)
    def _():
        pltpu.sync_copy(acc_shared.at[pl.ds(sid * NPW, NPW)],
                        out1_hbm.at[pl.ds(sid * NPW, NPW)])


@functools.cache
def _sc_scatter_kernel():
    return pl.kernel(
        _scatter_kernel_body,
        out_type=[
            jax.ShapeDtypeStruct((N_PAD, D), jnp.float32),
            jax.ShapeDtypeStruct((N_PAD, D), jnp.float32),
        ],
        mesh=_sc_mesh(),
        scratch_types=[
            pltpu.VMEM((SGROUP_ROWS, IDXB), jnp.int32),
            pltpu.VMEM((SGROUP_E, D), jnp.float32),
            pltpu.VMEM((ZROWS, D), jnp.float32),
            pltpu.VMEM_SHARED((N_PAD, D), jnp.float32),
        ],
        compiler_params=pltpu.CompilerParams(use_tc_tiling_on_sc=False),
    )


def _sc_scatter(eout, didx):
    return _sc_scatter_kernel()(eout, didx)


def kernel(node_states, edge_index, rel_idx, rel_emb, msg_W1, msg_b1,
           msg_W2, msg_b2, gate_W1, gate_b1, gate_W2, gate_b2,
           ln_gamma, ln_beta):
    L = msg_W1.shape[0]

    pad = E_PAD - E
    sidx = jnp.pad(edge_index[0], (0, pad)).reshape(E_PAD // IDXB, IDXB)
    didx = jnp.pad(edge_index[1], (0, pad)).reshape(E_PAD // IDXB, IDXB)
    rel3d = jnp.pad(rel_idx, (0, pad)).reshape(E_PAD // TE, 1, TE)

    h = node_states
    for l in range(L):
        wa = msg_W1[l, :D]
        ga = gate_W1[l, :D]
        rel1 = rel_emb @ msg_W1[l, D:] + msg_b1[l]
        relg = rel_emb @ gate_W1[l, 2 * D:] + gate_b1[l]
        reltab = jnp.zeros((8, 2 * D), jnp.float32)
        reltab = reltab.at[:NREL, :D].set(rel1).at[:NREL, D:].set(relg)
        w2 = msg_W2[l]
        b2 = msg_b2[l][None, :]
        g1b = gate_W1[l, D:2 * D]
        g2row = gate_W2[l][:, 0][None, :]
        gb2 = gate_b2[l][None, :]

        s, zd = _node_transform(h, wa, ga)
        src_g, dst_g = _sc_gather(s, zd, sidx, didx)
        eout = _edge_mlp(src_g, dst_g, rel3d, reltab, w2, b2, g1b, g2row,
                         gb2)
        p0, p1 = _sc_scatter(eout, didx)
        h = _residual_ln(h, p0, p1, ln_gamma[l][None, :],
                         ln_beta[l][None, :])
    return h


# speedup vs baseline: 2.5503x; 1.0772x over previous
import functools

import jax
import jax.numpy as jnp
from jax import lax
from jax.experimental import pallas as pl
from jax.experimental.pallas import tpu as pltpu
from jax.experimental.pallas import tpu_sc as plsc

N = 10000
D = 128
E = 320000
NREL = 6

NC = 2
NS = 16
NW = NC * NS
IDXB = 128
ROWS_PER_W = 80
EDGES_PER_W = ROWS_PER_W * IDXB
E_PAD = NW * EDGES_PER_W
GROUP_ROWS = 4
GROUP_E = GROUP_ROWS * IDXB
N_GROUPS = ROWS_PER_W // GROUP_ROWS
SGROUP_ROWS = 2
SGROUP_E = SGROUP_ROWS * IDXB
SN_GROUPS = ROWS_PER_W // SGROUP_ROWS
ZROWS = 64

TE = 2048
TN = 2000
N_PAD = 10240
NPW = N_PAD // NS


def _node_transform_body(h_ref, wa_ref, ga_ref, s_ref, zd_ref):
    x = h_ref[...]
    s_ref[...] = jnp.dot(x, wa_ref[...], preferred_element_type=jnp.float32)
    zd_ref[...] = jnp.dot(x, ga_ref[...], preferred_element_type=jnp.float32)


def _node_transform(h, wa, ga):
    return pl.pallas_call(
        _node_transform_body,
        grid=(N // TN,),
        in_specs=[
            pl.BlockSpec((TN, D), lambda i: (i, 0)),
            pl.BlockSpec((D, D), lambda i: (0, 0)),
            pl.BlockSpec((D, D), lambda i: (0, 0)),
        ],
        out_specs=[
            pl.BlockSpec((TN, D), lambda i: (i, 0)),
            pl.BlockSpec((TN, D), lambda i: (i, 0)),
        ],
        out_shape=[
            jax.ShapeDtypeStruct((N, D), jnp.float32),
            jax.ShapeDtypeStruct((N, D), jnp.float32),
        ],
    )(h, wa, ga)


def _gelu(x):
    return 0.5 * x * (1.0 + lax.erf(x * 0.7071067811865476))


def _edge_mlp_body(src_ref, dst_ref, rel_ref, reltab_ref, w2_ref, b2_ref,
                   g1b_ref, g2_ref, gb2_ref, out_ref):
    pid = pl.program_id(0)
    rel = rel_ref[0, 0, :]
    onehot = jnp.where(
        lax.broadcasted_iota(jnp.int32, (TE, 8), 1) == rel[:, None],
        1.0, 0.0).astype(jnp.float32)
    relb = jnp.dot(onehot, reltab_ref[...],
                   preferred_element_type=jnp.float32)
    hid = _gelu(src_ref[...] + relb[:, :D])
    msg = jnp.dot(hid, w2_ref[...],
                  preferred_element_type=jnp.float32) + b2_ref[...]
    g1 = dst_ref[...] + jnp.dot(msg, g1b_ref[...],
                                preferred_element_type=jnp.float32) \
        + relb[:, D:]
    ghid = _gelu(g1)
    glog = jnp.sum(ghid * g2_ref[...], axis=1, keepdims=True) + gb2_ref[0, 0]
    out = jax.nn.sigmoid(glog) * msg
    eid = pid * TE + lax.broadcasted_iota(jnp.int32, (TE, 1), 0)
    out_ref[...] = jnp.where(eid < E, out, 0.0)


def _edge_mlp(src_g, dst_g, rel3d, reltab, w2, b2, g1b, g2row, gb2):
    return pl.pallas_call(
        _edge_mlp_body,
        grid=(E_PAD // TE,),
        in_specs=[
            pl.BlockSpec((TE, D), lambda i: (i, 0)),
            pl.BlockSpec((TE, D), lambda i: (i, 0)),
            pl.BlockSpec((1, 1, TE), lambda i: (i, 0, 0)),
            pl.BlockSpec((8, 2 * D), lambda i: (0, 0)),
            pl.BlockSpec((D, D), lambda i: (0, 0)),
            pl.BlockSpec((1, D), lambda i: (0, 0)),
            pl.BlockSpec((D, D), lambda i: (0, 0)),
            pl.BlockSpec((1, D), lambda i: (0, 0)),
            pl.BlockSpec((1, 1), lambda i: (0, 0), memory_space=pltpu.SMEM),
        ],
        out_specs=pl.BlockSpec((TE, D), lambda i: (i, 0)),
        out_shape=jax.ShapeDtypeStruct((E_PAD, D), jnp.float32),
    )(src_g, dst_g, rel3d, reltab, w2, b2, g1b, g2row, gb2)


def _ln_body(h_ref, p0_ref, p1_ref, g_ref, b_ref, out_ref):
    x = h_ref[...] + p0_ref[...] + p1_ref[...]
    m = jnp.mean(x, axis=-1, keepdims=True)
    xc = x - m
    v = jnp.mean(xc * xc, axis=-1, keepdims=True)
    out_ref[...] = xc * lax.rsqrt(v + 1e-5) * g_ref[...] + b_ref[...]


def _residual_ln(h, p0, p1, gamma, beta):
    return pl.pallas_call(
        _ln_body,
        grid=(N // TN,),
        in_specs=[
            pl.BlockSpec((TN, D), lambda i: (i, 0)),
            pl.BlockSpec((TN, D), lambda i: (i, 0)),
            pl.BlockSpec((TN, D), lambda i: (i, 0)),
            pl.BlockSpec((1, D), lambda i: (0, 0)),
            pl.BlockSpec((1, D), lambda i: (0, 0)),
        ],
        out_specs=pl.BlockSpec((TN, D), lambda i: (i, 0)),
        out_shape=jax.ShapeDtypeStruct((N, D), jnp.float32),
    )(h, p0, p1, gamma, beta)


@functools.cache
def _sc_mesh():
    return plsc.VectorSubcoreMesh(core_axis_name="c", subcore_axis_name="s",
                                  num_cores=NC, num_subcores=NS)


KBUF = 5


def _gather_kernel_body(s_hbm, zd_hbm, sidx_hbm, didx_hbm, srcg_hbm,
                        dstg_hbm, idxall_v, rows_v, semg, semw):
    cid = lax.axis_index("c")
    sid = lax.axis_index("s")
    wid = sid * NC + cid
    row0 = wid * ROWS_PER_W
    e0 = wid * EDGES_PER_W

    def run(table_hbm, idx2d_hbm, out_hbm):
        pltpu.sync_copy(idx2d_hbm.at[pl.ds(row0, ROWS_PER_W)], idxall_v)

        def fire_gather(step, b):
            pltpu.async_copy(table_hbm.at[idxall_v.at[step]],
                             rows_v.at[b], semg.at[b])

        def wait_gather(b):
            pltpu.make_async_copy(table_hbm.at[pl.ds(0, IDXB)],
                                  rows_v.at[b], semg.at[b]).wait()

        def fire_wb(step, b):
            pltpu.async_copy(rows_v.at[b],
                             out_hbm.at[pl.ds(e0 + step * IDXB, IDXB)],
                             semw.at[b])

        def wait_wb(b):
            pltpu.make_async_copy(out_hbm.at[pl.ds(0, IDXB)],
                                  rows_v.at[b], semw.at[b]).wait()

        for b in range(KBUF):
            fire_gather(b, b)

        def body(k, carry):
            for j in range(KBUF):
                s = k * KBUF + j
                wait_gather(j)
                fire_wb(s, j)
                jp = (j - 1) % KBUF
                sp = s - 1

                @pl.when((sp >= 0) & (sp + KBUF < ROWS_PER_W))
                def _():
                    wait_wb(jp)
                    fire_gather(sp + KBUF, jp)
            return carry
        lax.fori_loop(0, ROWS_PER_W // KBUF, body, 0)

        for b in range(KBUF):
            wait_wb(b)

    run(s_hbm, sidx_hbm, srcg_hbm)
    run(zd_hbm, didx_hbm, dstg_hbm)


@functools.cache
def _sc_gather_kernel():
    return pl.kernel(
        _gather_kernel_body,
        out_type=[
            jax.ShapeDtypeStruct((E_PAD, D), jnp.float32),
            jax.ShapeDtypeStruct((E_PAD, D), jnp.float32),
        ],
        mesh=_sc_mesh(),
        scratch_types=[
            pltpu.VMEM((ROWS_PER_W, IDXB), jnp.int32),
            pltpu.VMEM((KBUF, IDXB, D), jnp.float32),
            pltpu.SemaphoreType.DMA((KBUF,)),
            pltpu.SemaphoreType.DMA((KBUF,)),
        ],
        compiler_params=pltpu.CompilerParams(use_tc_tiling_on_sc=False),
    )


def _sc_gather(s, zd, sidx, didx):
    return _sc_gather_kernel()(s, zd, sidx, didx)


def _scatter_kernel_body(eout_hbm, didx_hbm, out0_hbm, out1_hbm, idx_v,
                         vals_v, zbuf_v, acc_shared):
    cid = lax.axis_index("c")
    sid = lax.axis_index("s")
    wid = sid * NC + cid
    row0 = wid * ROWS_PER_W
    e0 = wid * EDGES_PER_W

    def zrow(i, carry):
        for j in range(D // 16):
            zbuf_v[i, pl.ds(j * 16, 16)] = jnp.zeros((16,), jnp.float32)
        return carry
    lax.fori_loop(0, ZROWS, zrow, 0)
    for t in range(NPW // ZROWS):
        pltpu.sync_copy(zbuf_v,
                        acc_shared.at[pl.ds(sid * NPW + t * ZROWS, ZROWS)])
    plsc.subcore_barrier()

    def group(g, carry):
        r = row0 + g * SGROUP_ROWS
        pltpu.sync_copy(didx_hbm.at[pl.ds(r, SGROUP_ROWS)], idx_v)
        pltpu.sync_copy(eout_hbm.at[pl.ds(e0 + g * SGROUP_E, SGROUP_E)],
                        vals_v)
        for j in range(SGROUP_ROWS):
            pltpu.sync_copy(vals_v.at[pl.ds(j * IDXB, IDXB)],
                            acc_shared.at[idx_v.at[j]], add=True)
        return carry
    lax.fori_loop(0, SN_GROUPS, group, 0)
    plsc.subcore_barrier()

    @pl.when(cid == 0)
    def _():
        pltpu.sync_copy(acc_shared.at[pl.ds(sid * NPW, NPW)],
                        out0_hbm.at[pl.ds(sid * NPW, NPW)])

    @pl.when(cid == 1)
    def _():
        pltpu.sync_copy(acc_shared.at[pl.ds(sid * NPW, NPW)],
                        out1_hbm.at[pl.ds(sid * NPW, NPW)])


@functools.cache
def _sc_scatter_kernel():
    return pl.kernel(
        _scatter_kernel_body,
        out_type=[
            jax.ShapeDtypeStruct((N_PAD, D), jnp.float32),
            jax.ShapeDtypeStruct((N_PAD, D), jnp.float32),
        ],
        mesh=_sc_mesh(),
        scratch_types=[
            pltpu.VMEM((SGROUP_ROWS, IDXB), jnp.int32),
            pltpu.VMEM((SGROUP_E, D), jnp.float32),
            pltpu.VMEM((ZROWS, D), jnp.float32),
            pltpu.VMEM_SHARED((N_PAD, D), jnp.float32),
        ],
        compiler_params=pltpu.CompilerParams(use_tc_tiling_on_sc=False),
    )


def _sc_scatter(eout, didx):
    return _sc_scatter_kernel()(eout, didx)


def kernel(node_states, edge_index, rel_idx, rel_emb, msg_W1, msg_b1,
           msg_W2, msg_b2, gate_W1, gate_b1, gate_W2, gate_b2,
           ln_gamma, ln_beta):
    L = msg_W1.shape[0]

    pad = E_PAD - E
    sidx = jnp.pad(edge_index[0], (0, pad)).reshape(E_PAD // IDXB, IDXB)
    didx = jnp.pad(edge_index[1], (0, pad)).reshape(E_PAD // IDXB, IDXB)
    rel3d = jnp.pad(rel_idx, (0, pad)).reshape(E_PAD // TE, 1, TE)

    h = node_states
    for l in range(L):
        wa = msg_W1[l, :D]
        ga = gate_W1[l, :D]
        rel1 = rel_emb @ msg_W1[l, D:] + msg_b1[l]
        relg = rel_emb @ gate_W1[l, 2 * D:] + gate_b1[l]
        reltab = jnp.zeros((8, 2 * D), jnp.float32)
        reltab = reltab.at[:NREL, :D].set(rel1).at[:NREL, D:].set(relg)
        w2 = msg_W2[l]
        b2 = msg_b2[l][None, :]
        g1b = gate_W1[l, D:2 * D]
        g2row = gate_W2[l][:, 0][None, :]
        gb2 = gate_b2[l][None, :]

        s, zd = _node_transform(h, wa, ga)
        src_g, dst_g = _sc_gather(s, zd, sidx, didx)
        eout = _edge_mlp(src_g, dst_g, rel3d, reltab, w2, b2, g1b, g2row,
                         gb2)
        p0, p1 = _sc_scatter(eout, didx)
        h = _residual_ln(h, p0, p1, ln_gamma[l][None, :],
                         ln_beta[l][None, :])
    return h


# speedup vs baseline: 2.6368x; 1.0339x over previous
import functools

import jax
import jax.numpy as jnp
from jax import lax
from jax.experimental import pallas as pl
from jax.experimental.pallas import tpu as pltpu
from jax.experimental.pallas import tpu_sc as plsc

N = 10000
D = 128
E = 320000
NREL = 6

NC = 2
NS = 16
NW = NC * NS
IDXB = 128
ROWS_PER_W = 80
EDGES_PER_W = ROWS_PER_W * IDXB
E_PAD = NW * EDGES_PER_W
GROUP_ROWS = 4
GROUP_E = GROUP_ROWS * IDXB
N_GROUPS = ROWS_PER_W // GROUP_ROWS
SGROUP_ROWS = 2
SGROUP_E = SGROUP_ROWS * IDXB
SN_GROUPS = ROWS_PER_W // SGROUP_ROWS
ZROWS = 64

TE = 2048
TN = 2000
N_PAD = 10240
NPW = N_PAD // NS


def _node_transform_body(h_ref, wa_ref, ga_ref, s_ref, zd_ref):
    x = h_ref[...]
    s_ref[...] = jnp.dot(x, wa_ref[...], preferred_element_type=jnp.float32)
    zd_ref[...] = jnp.dot(x, ga_ref[...], preferred_element_type=jnp.float32)


def _node_transform(h, wa, ga):
    return pl.pallas_call(
        _node_transform_body,
        grid=(N // TN,),
        in_specs=[
            pl.BlockSpec((TN, D), lambda i: (i, 0)),
            pl.BlockSpec((D, D), lambda i: (0, 0)),
            pl.BlockSpec((D, D), lambda i: (0, 0)),
        ],
        out_specs=[
            pl.BlockSpec((TN, D), lambda i: (i, 0)),
            pl.BlockSpec((TN, D), lambda i: (i, 0)),
        ],
        out_shape=[
            jax.ShapeDtypeStruct((N, D), jnp.float32),
            jax.ShapeDtypeStruct((N, D), jnp.float32),
        ],
    )(h, wa, ga)


def _gelu(x):
    return 0.5 * x * (1.0 + lax.erf(x * 0.7071067811865476))


def _edge_mlp_body(src_ref, dst_ref, rel_ref, reltab_ref, w2_ref, b2_ref,
                   g1b_ref, g2_ref, gb2_ref, out_ref):
    pid = pl.program_id(0)
    rel = rel_ref[0, 0, :]
    onehot = jnp.where(
        lax.broadcasted_iota(jnp.int32, (TE, 8), 1) == rel[:, None],
        1.0, 0.0).astype(jnp.float32)
    relb = jnp.dot(onehot, reltab_ref[...],
                   preferred_element_type=jnp.float32)
    hid = _gelu(src_ref[...] + relb[:, :D])
    msg = jnp.dot(hid, w2_ref[...],
                  preferred_element_type=jnp.float32) + b2_ref[...]
    g1 = dst_ref[...] + jnp.dot(msg, g1b_ref[...],
                                preferred_element_type=jnp.float32) \
        + relb[:, D:]
    ghid = _gelu(g1)
    glog = jnp.sum(ghid * g2_ref[...], axis=1, keepdims=True) + gb2_ref[0, 0]
    out = jax.nn.sigmoid(glog) * msg
    eid = pid * TE + lax.broadcasted_iota(jnp.int32, (TE, 1), 0)
    out_ref[...] = jnp.where(eid < E, out, 0.0)


def _edge_mlp(src_g, dst_g, rel3d, reltab, w2, b2, g1b, g2row, gb2):
    return pl.pallas_call(
        _edge_mlp_body,
        grid=(E_PAD // TE,),
        in_specs=[
            pl.BlockSpec((TE, D), lambda i: (i, 0)),
            pl.BlockSpec((TE, D), lambda i: (i, 0)),
            pl.BlockSpec((1, 1, TE), lambda i: (i, 0, 0)),
            pl.BlockSpec((8, 2 * D), lambda i: (0, 0)),
            pl.BlockSpec((D, D), lambda i: (0, 0)),
            pl.BlockSpec((1, D), lambda i: (0, 0)),
            pl.BlockSpec((D, D), lambda i: (0, 0)),
            pl.BlockSpec((1, D), lambda i: (0, 0)),
            pl.BlockSpec((1, 1), lambda i: (0, 0), memory_space=pltpu.SMEM),
        ],
        out_specs=pl.BlockSpec((TE, D), lambda i: (i, 0)),
        out_shape=jax.ShapeDtypeStruct((E_PAD, D), jnp.float32),
    )(src_g, dst_g, rel3d, reltab, w2, b2, g1b, g2row, gb2)


def _ln_body(h_ref, p0_ref, p1_ref, g_ref, b_ref, out_ref):
    x = h_ref[...] + p0_ref[...] + p1_ref[...]
    m = jnp.mean(x, axis=-1, keepdims=True)
    xc = x - m
    v = jnp.mean(xc * xc, axis=-1, keepdims=True)
    out_ref[...] = xc * lax.rsqrt(v + 1e-5) * g_ref[...] + b_ref[...]


def _residual_ln(h, p0, p1, gamma, beta):
    return pl.pallas_call(
        _ln_body,
        grid=(N // TN,),
        in_specs=[
            pl.BlockSpec((TN, D), lambda i: (i, 0)),
            pl.BlockSpec((TN, D), lambda i: (i, 0)),
            pl.BlockSpec((TN, D), lambda i: (i, 0)),
            pl.BlockSpec((1, D), lambda i: (0, 0)),
            pl.BlockSpec((1, D), lambda i: (0, 0)),
        ],
        out_specs=pl.BlockSpec((TN, D), lambda i: (i, 0)),
        out_shape=jax.ShapeDtypeStruct((N, D), jnp.float32),
    )(h, p0, p1, gamma, beta)


@functools.cache
def _sc_mesh():
    return plsc.VectorSubcoreMesh(core_axis_name="c", subcore_axis_name="s",
                                  num_cores=NC, num_subcores=NS)


KBUF = 5

FAST_CID = 1
ROWS_FAST = 130
ROWS_SLOW = 30
assert NS * (ROWS_FAST + ROWS_SLOW) == E_PAD // IDXB


def _gather_kernel_body(s_hbm, zd_hbm, sidx_hbm, didx_hbm, srcg_hbm,
                        dstg_hbm, idxall_v, rows_v, semg, semw):
    cid = lax.axis_index("c")
    sid = lax.axis_index("s")
    is_fast = cid == FAST_CID
    row0 = jnp.where(is_fast, sid * ROWS_FAST,
                     NS * ROWS_FAST + sid * ROWS_SLOW)
    nrows = jnp.where(is_fast, ROWS_FAST, ROWS_SLOW)
    e0 = row0 * IDXB

    def run(table_hbm, idx2d_hbm, out_hbm):
        pltpu.sync_copy(idx2d_hbm.at[pl.ds(row0, ROWS_FAST)], idxall_v)

        def fire_gather(step, b):
            pltpu.async_copy(table_hbm.at[idxall_v.at[step]],
                             rows_v.at[b], semg.at[b])

        def wait_gather(b):
            pltpu.make_async_copy(table_hbm.at[pl.ds(0, IDXB)],
                                  rows_v.at[b], semg.at[b]).wait()

        def fire_wb(step, b):
            pltpu.async_copy(rows_v.at[b],
                             out_hbm.at[pl.ds(e0 + step * IDXB, IDXB)],
                             semw.at[b])

        def wait_wb(b):
            pltpu.make_async_copy(out_hbm.at[pl.ds(0, IDXB)],
                                  rows_v.at[b], semw.at[b]).wait()

        for b in range(KBUF):
            fire_gather(b, b)

        def body(k, carry):
            for j in range(KBUF):
                s = k * KBUF + j
                wait_gather(j)
                fire_wb(s, j)
                jp = (j - 1) % KBUF
                sp = s - 1

                @pl.when((sp >= 0) & (sp + KBUF < nrows))
                def _():
                    wait_wb(jp)
                    fire_gather(sp + KBUF, jp)
            return carry
        lax.fori_loop(0, nrows // KBUF, body, 0)

        for b in range(KBUF):
            wait_wb(b)

    run(s_hbm, sidx_hbm, srcg_hbm)
    run(zd_hbm, didx_hbm, dstg_hbm)


@functools.cache
def _sc_gather_kernel():
    return pl.kernel(
        _gather_kernel_body,
        out_type=[
            jax.ShapeDtypeStruct((E_PAD, D), jnp.float32),
            jax.ShapeDtypeStruct((E_PAD, D), jnp.float32),
        ],
        mesh=_sc_mesh(),
        scratch_types=[
            pltpu.VMEM((ROWS_FAST, IDXB), jnp.int32),
            pltpu.VMEM((KBUF, IDXB, D), jnp.float32),
            pltpu.SemaphoreType.DMA((KBUF,)),
            pltpu.SemaphoreType.DMA((KBUF,)),
        ],
        compiler_params=pltpu.CompilerParams(use_tc_tiling_on_sc=False),
    )


def _sc_gather(s, zd, sidx, didx):
    return _sc_gather_kernel()(s, zd, sidx, didx)


def _scatter_kernel_body(eout_hbm, didx_hbm, out0_hbm, out1_hbm, idx_v,
                         vals_v, zbuf_v, acc_shared):
    cid = lax.axis_index("c")
    sid = lax.axis_index("s")
    wid = sid * NC + cid
    row0 = wid * ROWS_PER_W
    e0 = wid * EDGES_PER_W

    def zrow(i, carry):
        for j in range(D // 16):
            zbuf_v[i, pl.ds(j * 16, 16)] = jnp.zeros((16,), jnp.float32)
        return carry
    lax.fori_loop(0, ZROWS, zrow, 0)
    for t in range(NPW // ZROWS):
        pltpu.sync_copy(zbuf_v,
                        acc_shared.at[pl.ds(sid * NPW + t * ZROWS, ZROWS)])
    plsc.subcore_barrier()

    def group(g, carry):
        r = row0 + g * SGROUP_ROWS
        pltpu.sync_copy(didx_hbm.at[pl.ds(r, SGROUP_ROWS)], idx_v)
        pltpu.sync_copy(eout_hbm.at[pl.ds(e0 + g * SGROUP_E, SGROUP_E)],
                        vals_v)
        for j in range(SGROUP_ROWS):
            pltpu.sync_copy(vals_v.at[pl.ds(j * IDXB, IDXB)],
                            acc_shared.at[idx_v.at[j]], add=True)
        return carry
    lax.fori_loop(0, SN_GROUPS, group, 0)
    plsc.subcore_barrier()

    @pl.when(cid == 0)
    def _():
        pltpu.sync_copy(acc_shared.at[pl.ds(sid * NPW, NPW)],
                        out0_hbm.at[pl.ds(sid * NPW, NPW)])

    @pl.when(cid == 1)
    def _():
        pltpu.sync_copy(acc_shared.at[pl.ds(sid * NPW, NPW)],
                        out1_hbm.at[pl.ds(sid * NPW, NPW)])


@functools.cache
def _sc_scatter_kernel():
    return pl.kernel(
        _scatter_kernel_body,
        out_type=[
            jax.ShapeDtypeStruct((N_PAD, D), jnp.float32),
            jax.ShapeDtypeStruct((N_PAD, D), jnp.float32),
        ],
        mesh=_sc_mesh(),
        scratch_types=[
            pltpu.VMEM((SGROUP_ROWS, IDXB), jnp.int32),
            pltpu.VMEM((SGROUP_E, D), jnp.float32),
            pltpu.VMEM((ZROWS, D), jnp.float32),
            pltpu.VMEM_SHARED((N_PAD, D), jnp.float32),
        ],
        compiler_params=pltpu.CompilerParams(use_tc_tiling_on_sc=False),
    )


def _sc_scatter(eout, didx):
    return _sc_scatter_kernel()(eout, didx)


def kernel(node_states, edge_index, rel_idx, rel_emb, msg_W1, msg_b1,
           msg_W2, msg_b2, gate_W1, gate_b1, gate_W2, gate_b2,
           ln_gamma, ln_beta):
    L = msg_W1.shape[0]

    pad = E_PAD - E
    sidx = jnp.pad(edge_index[0], (0, pad)).reshape(E_PAD // IDXB, IDXB)
    sidx = jnp.pad(sidx, ((0, ROWS_FAST), (0, 0)))
    didx = jnp.pad(edge_index[1], (0, pad)).reshape(E_PAD // IDXB, IDXB)
    didx = jnp.pad(didx, ((0, ROWS_FAST), (0, 0)))
    rel3d = jnp.pad(rel_idx, (0, pad)).reshape(E_PAD // TE, 1, TE)

    h = node_states
    for l in range(L):
        wa = msg_W1[l, :D]
        ga = gate_W1[l, :D]
        rel1 = rel_emb @ msg_W1[l, D:] + msg_b1[l]
        relg = rel_emb @ gate_W1[l, 2 * D:] + gate_b1[l]
        reltab = jnp.zeros((8, 2 * D), jnp.float32)
        reltab = reltab.at[:NREL, :D].set(rel1).at[:NREL, D:].set(relg)
        w2 = msg_W2[l]
        b2 = msg_b2[l][None, :]
        g1b = gate_W1[l, D:2 * D]
        g2row = gate_W2[l][:, 0][None, :]
        gb2 = gate_b2[l][None, :]

        s, zd = _node_transform(h, wa, ga)
        src_g, dst_g = _sc_gather(s, zd, sidx, didx)
        eout = _edge_mlp(src_g, dst_g, rel3d, reltab, w2, b2, g1b, g2row,
                         gb2)
        p0, p1 = _sc_scatter(eout, didx)
        h = _residual_ln(h, p0, p1, ln_gamma[l][None, :],
                         ln_beta[l][None, :])
    return h
